# baseline (speedup 1.0000x reference)
import functools

import jax
import jax.numpy as jnp
from jax import lax
from jax.experimental import pallas as pl
from jax.experimental.pallas import tpu as pltpu
from jax.experimental.pallas import tpu_sc as plsc

_NC = 2
_NS = 16
_NW = _NC * _NS

_BATCH = 16384
_D = 128
_CHUNK = 64


def _sc_gather_body(base_hbm, uidx_hbm, iidx_hbm, t_umf, t_imf, t_umlp,
                    t_imlp, wo_hbm,
                    o_umlp, o_imlp, o_s,
                    base_v, uidx_v, iidx_v, wo_v, sdot_v,
                    mf_u0, mf_u1, mf_i0, mf_i1,
                    ml_u0, ml_u1, ml_i0, ml_i1,
                    s_mfu0, s_mfu1, s_mfi0, s_mfi1,
                    s_mlu0, s_mlu1, s_mli0, s_mli1,
                    s_wbu0, s_wbu1, s_wbi0, s_wbi1,
                    batch=None):
  rows_per_w = batch // _NW
  n_chunks = rows_per_w // _CHUNK
  wid = lax.axis_index("s") * _NC + lax.axis_index("c")
  out_base = wid * rows_per_w

  pltpu.sync_copy(base_hbm, base_v)
  base = pl.multiple_of(base_v[...][0], 256)
  pltpu.sync_copy(uidx_hbm.at[pl.ds(base + out_base, rows_per_w)], uidx_v)
  pltpu.sync_copy(iidx_hbm.at[pl.ds(base + out_base, rows_per_w)], iidx_v)
  pltpu.sync_copy(wo_hbm, wo_v)
  w_regs = [wo_v[0, pl.ds(16 * j, 16)] for j in range(_D // 16)]

  def uidx(c):
    return uidx_v.at[pl.ds(c * _CHUNK, _CHUNK)]

  def iidx(c):
    return iidx_v.at[pl.ds(c * _CHUNK, _CHUNK)]

  mf_u, mf_i = (mf_u0, mf_u1), (mf_i0, mf_i1)
  ml_u, ml_i = (ml_u0, ml_u1), (ml_i0, ml_i1)
  smf_u, smf_i = (s_mfu0, s_mfu1), (s_mfi0, s_mfi1)
  sml_u, sml_i = (s_mlu0, s_mlu1), (s_mli0, s_mli1)
  swb_u, swb_i = (s_wbu0, s_wbu1), (s_wbi0, s_wbi1)

  def start(c):
    p = c % 2
    pltpu.async_copy(t_umf.at[uidx(c)], mf_u[p], smf_u[p])
    pltpu.async_copy(t_imf.at[iidx(c)], mf_i[p], smf_i[p])
    pltpu.async_copy(t_umlp.at[uidx(c)], ml_u[p], sml_u[p])
    pltpu.async_copy(t_imlp.at[iidx(c)], ml_i[p], sml_i[p])

  start(0)
  for c in range(n_chunks):
    p = c % 2
    if c + 1 < n_chunks:
      if c >= 1:
        pltpu.make_async_copy(
            ml_u[1 - p], o_umlp.at[pl.ds(out_base + (c - 1) * _CHUNK, _CHUNK)],
            swb_u[1 - p]).wait()
        pltpu.make_async_copy(
            ml_i[1 - p], o_imlp.at[pl.ds(out_base + (c - 1) * _CHUNK, _CHUNK)],
            swb_i[1 - p]).wait()
      start(c + 1)
    pltpu.make_async_copy(t_umlp.at[uidx(c)], ml_u[p], sml_u[p]).wait()
    pltpu.make_async_copy(t_imlp.at[iidx(c)], ml_i[p], sml_i[p]).wait()
    pltpu.async_copy(
        ml_u[p], o_umlp.at[pl.ds(out_base + c * _CHUNK, _CHUNK)], swb_u[p])
    pltpu.async_copy(
        ml_i[p], o_imlp.at[pl.ds(out_base + c * _CHUNK, _CHUNK)], swb_i[p])

    pltpu.make_async_copy(t_umf.at[uidx(c)], mf_u[p], smf_u[p]).wait()
    pltpu.make_async_copy(t_imf.at[iidx(c)], mf_i[p], smf_i[p]).wait()
    ub, ib = mf_u[p], mf_i[p]

    def row_body(r, _, ub=ub, ib=ib, c=c):
      acc = ub[r, pl.ds(0, 16)] * ib[r, pl.ds(0, 16)] * w_regs[0]
      for j in range(1, _D // 16):
        acc += ub[r, pl.ds(16 * j, 16)] * ib[r, pl.ds(16 * j, 16)] * w_regs[j]
      sdot_v[c * _CHUNK + r, :] = acc
      return 0

    lax.fori_loop(0, _CHUNK, row_body, 0)

  for c in (n_chunks - 2, n_chunks - 1):
    p = c % 2
    pltpu.make_async_copy(
        ml_u[p], o_umlp.at[pl.ds(out_base + c * _CHUNK, _CHUNK)],
        swb_u[p]).wait()
    pltpu.make_async_copy(
        ml_i[p], o_imlp.at[pl.ds(out_base + c * _CHUNK, _CHUNK)],
        swb_i[p]).wait()

  pltpu.sync_copy(sdot_v, o_s.at[pl.ds(out_base, rows_per_w)])


def _sc_gather(base, uidx, iidx, t_umf, t_imf, t_umlp, t_imlp, Wo, batch):
  rows_per_w = batch // _NW
  mesh = plsc.VectorSubcoreMesh(core_axis_name="c", subcore_axis_name="s",
                                num_cores=_NC, num_subcores=_NS)
  out = jax.ShapeDtypeStruct((batch, _D), jnp.float32)
  out_s = jax.ShapeDtypeStruct((batch, 16), jnp.float32)
  buf = pltpu.VMEM((_CHUNK, _D), jnp.float32)
  k = pl.kernel(
      functools.partial(_sc_gather_body, batch=batch),
      out_type=(out, out, out_s),
      mesh=mesh,
      scratch_types=[
          pltpu.VMEM((16,), jnp.int32),
          pltpu.VMEM((rows_per_w,), jnp.int32),
          pltpu.VMEM((rows_per_w,), jnp.int32),
          pltpu.VMEM((1, _D + 32), jnp.float32),
          pltpu.VMEM((rows_per_w, 16), jnp.float32),
      ] + [buf] * 8 + [pltpu.SemaphoreType.DMA] * 12,
  )
  return k(base, uidx, iidx, t_umf, t_imf, t_umlp, t_imlp, Wo)


_BLK = 1024


def _tc_mlp_body(umlp, imlp, s, w1, b1, w2, b2, w3, b3, wo, bo, out):
  cT = (((1,), (1,)), ((), ()))
  cN = (((1,), (0,)), ((), ()))
  h = lax.dot_general(w1[:, :_D], umlp[...], cT,
                      preferred_element_type=jnp.float32)
  h += lax.dot_general(w1[:, _D:], imlp[...], cT,
                       preferred_element_type=jnp.float32)
  h = jnp.maximum(h + b1[...][:, None], 0.0)
  h = jnp.maximum(
      lax.dot_general(w2[...], h, cN, preferred_element_type=jnp.float32)
      + b2[...][:, None], 0.0)
  h = jnp.maximum(
      lax.dot_general(w3[...], h, cN, preferred_element_type=jnp.float32)
      + b3[...][:, None], 0.0)
  logit = lax.dot_general(wo[:, _D:], h, cN,
                          preferred_element_type=jnp.float32)
  ones16 = jnp.ones((1, 16), dtype=jnp.float32)
  logit += lax.dot_general(ones16, s[...], cT,
                           preferred_element_type=jnp.float32)
  logit += bo[0]
  out[...] = (1.0 / (1.0 + jnp.exp(-logit)))[0, :]


def _tc_mlp(umlp, imlp, s, W1, b1, W2, b2, W3, b3, Wo, bo, batch):
  n_blk = batch // _BLK
  batch_spec = pl.BlockSpec((_BLK, _D), lambda i: (i, 0))
  s_spec = pl.BlockSpec((_BLK, 16), lambda i: (i, 0))
  full = lambda shape: pl.BlockSpec(shape, lambda i: tuple(0 for _ in shape))
  return pl.pallas_call(
      _tc_mlp_body,
      grid=(n_blk,),
      in_specs=[
          batch_spec, batch_spec, s_spec,
          full((_D, 2 * _D)), full((_D,)),
          full((64, _D)), full((64,)),
          full((32, 64)), full((32,)),
          full((1, _D + 32)), full((1,)),
      ],
      out_specs=pl.BlockSpec((_BLK,), lambda i: (i,)),
      out_shape=jax.ShapeDtypeStruct((batch,), jnp.float32),
  )(umlp, imlp, s, W1, b1, W2, b2, W3, b3, Wo, bo)


_N_SPLIT = 2


@jax.jit
def kernel(user_indices, item_indices, user_emb_mf, item_emb_mf,
           user_emb_mlp, item_emb_mlp, W1, b1, W2, b2, W3, b3, Wo, bo):
  half = _BATCH // _N_SPLIT
  outs = []
  for h in range(_N_SPLIT):
    base = jnp.full((16,), h * half, dtype=jnp.int32)
    umlp, imlp, s = _sc_gather(base, user_indices, item_indices, user_emb_mf,
                               item_emb_mf, user_emb_mlp, item_emb_mlp, Wo,
                               half)
    outs.append(_tc_mlp(umlp, imlp, s, W1, b1, W2, b2, W3, b3, Wo, bo, half))
  return jnp.concatenate(outs)


# speedup vs baseline: 1.0101x; 1.0101x over previous
import functools

import jax
import jax.numpy as jnp
from jax import lax
from jax.experimental import pallas as pl
from jax.experimental.pallas import tpu as pltpu
from jax.experimental.pallas import tpu_sc as plsc

_NC = 2
_NS = 16
_NW = _NC * _NS

_BATCH = 16384
_D = 128
_CHUNK = 64


def _sc_gather_body(base_hbm, uidx_hbm, iidx_hbm, t_umf, t_imf, t_umlp,
                    t_imlp, wo_hbm,
                    o_umlp, o_imlp, o_s,
                    base_v, uidx_v, iidx_v, wo_v, sdot_v,
                    mf_u0, mf_u1, mf_i0, mf_i1,
                    ml_u0, ml_u1, ml_i0, ml_i1,
                    s_mfu0, s_mfu1, s_mfi0, s_mfi1,
                    s_mlu0, s_mlu1, s_mli0, s_mli1,
                    s_wbu0, s_wbu1, s_wbi0, s_wbi1,
                    batch=None):
  rows_per_w = batch // _NW
  n_chunks = rows_per_w // _CHUNK
  wid = lax.axis_index("s") * _NC + lax.axis_index("c")
  out_base = wid * rows_per_w

  pltpu.sync_copy(base_hbm, base_v)
  base = pl.multiple_of(base_v[...][0], 256)
  pltpu.sync_copy(uidx_hbm.at[pl.ds(base + out_base, rows_per_w)], uidx_v)
  pltpu.sync_copy(iidx_hbm.at[pl.ds(base + out_base, rows_per_w)], iidx_v)
  pltpu.sync_copy(wo_hbm, wo_v)
  w_regs = [wo_v[0, pl.ds(16 * j, 16)] for j in range(_D // 16)]

  def uidx(c):
    return uidx_v.at[pl.ds(c * _CHUNK, _CHUNK)]

  def iidx(c):
    return iidx_v.at[pl.ds(c * _CHUNK, _CHUNK)]

  mf_u, mf_i = (mf_u0, mf_u1), (mf_i0, mf_i1)
  ml_u, ml_i = (ml_u0, ml_u1), (ml_i0, ml_i1)
  smf_u, smf_i = (s_mfu0, s_mfu1), (s_mfi0, s_mfi1)
  sml_u, sml_i = (s_mlu0, s_mlu1), (s_mli0, s_mli1)
  swb_u, swb_i = (s_wbu0, s_wbu1), (s_wbi0, s_wbi1)

  def start(c):
    p = c % 2
    pltpu.async_copy(t_umf.at[uidx(c)], mf_u[p], smf_u[p])
    pltpu.async_copy(t_imf.at[iidx(c)], mf_i[p], smf_i[p])
    pltpu.async_copy(t_umlp.at[uidx(c)], ml_u[p], sml_u[p])
    pltpu.async_copy(t_imlp.at[iidx(c)], ml_i[p], sml_i[p])

  start(0)
  for c in range(n_chunks):
    p = c % 2
    if c + 1 < n_chunks:
      if c >= 1:
        pltpu.make_async_copy(
            ml_u[1 - p], o_umlp.at[pl.ds(out_base + (c - 1) * _CHUNK, _CHUNK)],
            swb_u[1 - p]).wait()
        pltpu.make_async_copy(
            ml_i[1 - p], o_imlp.at[pl.ds(out_base + (c - 1) * _CHUNK, _CHUNK)],
            swb_i[1 - p]).wait()
      start(c + 1)
    pltpu.make_async_copy(t_umlp.at[uidx(c)], ml_u[p], sml_u[p]).wait()
    pltpu.make_async_copy(t_imlp.at[iidx(c)], ml_i[p], sml_i[p]).wait()
    pltpu.async_copy(
        ml_u[p], o_umlp.at[pl.ds(out_base + c * _CHUNK, _CHUNK)], swb_u[p])
    pltpu.async_copy(
        ml_i[p], o_imlp.at[pl.ds(out_base + c * _CHUNK, _CHUNK)], swb_i[p])

    pltpu.make_async_copy(t_umf.at[uidx(c)], mf_u[p], smf_u[p]).wait()
    pltpu.make_async_copy(t_imf.at[iidx(c)], mf_i[p], smf_i[p]).wait()
    ub, ib = mf_u[p], mf_i[p]

    def row_body(r, _, ub=ub, ib=ib, c=c):
      acc = ub[r, pl.ds(0, 16)] * ib[r, pl.ds(0, 16)] * w_regs[0]
      for j in range(1, _D // 16):
        acc += ub[r, pl.ds(16 * j, 16)] * ib[r, pl.ds(16 * j, 16)] * w_regs[j]
      sdot_v[c * _CHUNK + r, :] = acc
      return 0

    lax.fori_loop(0, _CHUNK, row_body, 0)

  for c in (n_chunks - 2, n_chunks - 1):
    p = c % 2
    pltpu.make_async_copy(
        ml_u[p], o_umlp.at[pl.ds(out_base + c * _CHUNK, _CHUNK)],
        swb_u[p]).wait()
    pltpu.make_async_copy(
        ml_i[p], o_imlp.at[pl.ds(out_base + c * _CHUNK, _CHUNK)],
        swb_i[p]).wait()

  pltpu.sync_copy(sdot_v, o_s.at[pl.ds(out_base, rows_per_w)])


@functools.lru_cache(maxsize=None)
def _sc_gather_kernel(batch):
  rows_per_w = batch // _NW
  mesh = plsc.VectorSubcoreMesh(core_axis_name="c", subcore_axis_name="s",
                                num_cores=_NC, num_subcores=_NS)
  out = jax.ShapeDtypeStruct((batch, _D), jnp.float32)
  out_s = jax.ShapeDtypeStruct((batch, 16), jnp.float32)
  buf = pltpu.VMEM((_CHUNK, _D), jnp.float32)
  k = pl.kernel(
      functools.partial(_sc_gather_body, batch=batch),
      out_type=(out, out, out_s),
      mesh=mesh,
      scratch_types=[
          pltpu.VMEM((16,), jnp.int32),
          pltpu.VMEM((rows_per_w,), jnp.int32),
          pltpu.VMEM((rows_per_w,), jnp.int32),
          pltpu.VMEM((1, _D + 32), jnp.float32),
          pltpu.VMEM((rows_per_w, 16), jnp.float32),
      ] + [buf] * 8 + [pltpu.SemaphoreType.DMA] * 12,
  )
  return k


def _sc_gather(base, uidx, iidx, t_umf, t_imf, t_umlp, t_imlp, Wo, batch):
  return _sc_gather_kernel(batch)(base, uidx, iidx, t_umf, t_imf, t_umlp,
                                  t_imlp, Wo)


_BLK = 1024


def _tc_mlp_body(umlp, imlp, s, w1, b1, w2, b2, w3, b3, wo, bo, out):
  cT = (((1,), (1,)), ((), ()))
  cN = (((1,), (0,)), ((), ()))
  h = lax.dot_general(w1[:, :_D], umlp[...], cT,
                      preferred_element_type=jnp.float32)
  h += lax.dot_general(w1[:, _D:], imlp[...], cT,
                       preferred_element_type=jnp.float32)
  h = jnp.maximum(h + b1[...][:, None], 0.0)
  h = jnp.maximum(
      lax.dot_general(w2[...], h, cN, preferred_element_type=jnp.float32)
      + b2[...][:, None], 0.0)
  h = jnp.maximum(
      lax.dot_general(w3[...], h, cN, preferred_element_type=jnp.float32)
      + b3[...][:, None], 0.0)
  logit = lax.dot_general(wo[:, _D:], h, cN,
                          preferred_element_type=jnp.float32)
  ones16 = jnp.ones((1, 16), dtype=jnp.float32)
  logit += lax.dot_general(ones16, s[...], cT,
                           preferred_element_type=jnp.float32)
  logit += bo[0]
  out[...] = (1.0 / (1.0 + jnp.exp(-logit)))[0, :]


def _tc_mlp(umlp, imlp, s, W1, b1, W2, b2, W3, b3, Wo, bo, batch):
  n_blk = batch // _BLK
  batch_spec = pl.BlockSpec((_BLK, _D), lambda i: (i, 0))
  s_spec = pl.BlockSpec((_BLK, 16), lambda i: (i, 0))
  full = lambda shape: pl.BlockSpec(shape, lambda i: tuple(0 for _ in shape))
  return pl.pallas_call(
      _tc_mlp_body,
      grid=(n_blk,),
      in_specs=[
          batch_spec, batch_spec, s_spec,
          full((_D, 2 * _D)), full((_D,)),
          full((64, _D)), full((64,)),
          full((32, 64)), full((32,)),
          full((1, _D + 32)), full((1,)),
      ],
      out_specs=pl.BlockSpec((_BLK,), lambda i: (i,)),
      out_shape=jax.ShapeDtypeStruct((batch,), jnp.float32),
  )(umlp, imlp, s, W1, b1, W2, b2, W3, b3, Wo, bo)


_N_SPLIT = 2


@jax.jit
def kernel(user_indices, item_indices, user_emb_mf, item_emb_mf,
           user_emb_mlp, item_emb_mlp, W1, b1, W2, b2, W3, b3, Wo, bo):
  half = _BATCH // _N_SPLIT
  outs = []
  for h in range(_N_SPLIT):
    base = jnp.full((16,), h * half, dtype=jnp.int32)
    umlp, imlp, s = _sc_gather(base, user_indices, item_indices, user_emb_mf,
                               item_emb_mf, user_emb_mlp, item_emb_mlp, Wo,
                               half)
    outs.append(_tc_mlp(umlp, imlp, s, W1, b1, W2, b2, W3, b3, Wo, bo, half))
  return jnp.concatenate(outs)


# speedup vs baseline: 1.0134x; 1.0033x over previous
import functools

import jax
import jax.numpy as jnp
from jax import lax
from jax.experimental import pallas as pl
from jax.experimental.pallas import tpu as pltpu
from jax.experimental.pallas import tpu_sc as plsc

_NC = 2
_NS = 16
_NW = _NC * _NS

_BATCH = 16384
_D = 128
_CHUNK = 32


def _sc_gather_body(base_hbm, uidx_hbm, iidx_hbm, t_umf, t_imf, t_umlp,
                    t_imlp, wo_hbm,
                    o_umlp, o_imlp, o_s,
                    base_v, uidx_v, iidx_v, wo_v, sdot_v,
                    mf_u0, mf_u1, mf_i0, mf_i1,
                    ml_u0, ml_u1, ml_i0, ml_i1,
                    s_mfu0, s_mfu1, s_mfi0, s_mfi1,
                    s_mlu0, s_mlu1, s_mli0, s_mli1,
                    s_wbu0, s_wbu1, s_wbi0, s_wbi1,
                    batch=None):
  rows_per_w = batch // _NW
  n_chunks = rows_per_w // _CHUNK
  wid = lax.axis_index("s") * _NC + lax.axis_index("c")
  out_base = wid * rows_per_w

  pltpu.sync_copy(base_hbm, base_v)
  base = pl.multiple_of(base_v[...][0], 256)
  pltpu.sync_copy(uidx_hbm.at[pl.ds(base + out_base, rows_per_w)], uidx_v)
  pltpu.sync_copy(iidx_hbm.at[pl.ds(base + out_base, rows_per_w)], iidx_v)
  pltpu.sync_copy(wo_hbm, wo_v)
  w_regs = [wo_v[0, pl.ds(16 * j, 16)] for j in range(_D // 16)]

  def uidx(c):
    return uidx_v.at[pl.ds(c * _CHUNK, _CHUNK)]

  def iidx(c):
    return iidx_v.at[pl.ds(c * _CHUNK, _CHUNK)]

  mf_u, mf_i = (mf_u0, mf_u1), (mf_i0, mf_i1)
  ml_u, ml_i = (ml_u0, ml_u1), (ml_i0, ml_i1)
  smf_u, smf_i = (s_mfu0, s_mfu1), (s_mfi0, s_mfi1)
  sml_u, sml_i = (s_mlu0, s_mlu1), (s_mli0, s_mli1)
  swb_u, swb_i = (s_wbu0, s_wbu1), (s_wbi0, s_wbi1)

  def start(c):
    p = c % 2
    pltpu.async_copy(t_umf.at[uidx(c)], mf_u[p], smf_u[p])
    pltpu.async_copy(t_imf.at[iidx(c)], mf_i[p], smf_i[p])
    pltpu.async_copy(t_umlp.at[uidx(c)], ml_u[p], sml_u[p])
    pltpu.async_copy(t_imlp.at[iidx(c)], ml_i[p], sml_i[p])

  start(0)
  for c in range(n_chunks):
    p = c % 2
    if c + 1 < n_chunks:
      if c >= 1:
        pltpu.make_async_copy(
            ml_u[1 - p], o_umlp.at[pl.ds(out_base + (c - 1) * _CHUNK, _CHUNK)],
            swb_u[1 - p]).wait()
        pltpu.make_async_copy(
            ml_i[1 - p], o_imlp.at[pl.ds(out_base + (c - 1) * _CHUNK, _CHUNK)],
            swb_i[1 - p]).wait()
      start(c + 1)
    pltpu.make_async_copy(t_umlp.at[uidx(c)], ml_u[p], sml_u[p]).wait()
    pltpu.make_async_copy(t_imlp.at[iidx(c)], ml_i[p], sml_i[p]).wait()
    pltpu.async_copy(
        ml_u[p], o_umlp.at[pl.ds(out_base + c * _CHUNK, _CHUNK)], swb_u[p])
    pltpu.async_copy(
        ml_i[p], o_imlp.at[pl.ds(out_base + c * _CHUNK, _CHUNK)], swb_i[p])

    pltpu.make_async_copy(t_umf.at[uidx(c)], mf_u[p], smf_u[p]).wait()
    pltpu.make_async_copy(t_imf.at[iidx(c)], mf_i[p], smf_i[p]).wait()
    ub, ib = mf_u[p], mf_i[p]

    def row_body(r, _, ub=ub, ib=ib, c=c):
      acc = ub[r, pl.ds(0, 16)] * ib[r, pl.ds(0, 16)] * w_regs[0]
      for j in range(1, _D // 16):
        acc += ub[r, pl.ds(16 * j, 16)] * ib[r, pl.ds(16 * j, 16)] * w_regs[j]
      sdot_v[c * _CHUNK + r, :] = acc
      return 0

    lax.fori_loop(0, _CHUNK, row_body, 0)

  for c in (n_chunks - 2, n_chunks - 1):
    p = c % 2
    pltpu.make_async_copy(
        ml_u[p], o_umlp.at[pl.ds(out_base + c * _CHUNK, _CHUNK)],
        swb_u[p]).wait()
    pltpu.make_async_copy(
        ml_i[p], o_imlp.at[pl.ds(out_base + c * _CHUNK, _CHUNK)],
        swb_i[p]).wait()

  pltpu.sync_copy(sdot_v, o_s.at[pl.ds(out_base, rows_per_w)])


@functools.lru_cache(maxsize=None)
def _sc_gather_kernel(batch):
  rows_per_w = batch // _NW
  mesh = plsc.VectorSubcoreMesh(core_axis_name="c", subcore_axis_name="s",
                                num_cores=_NC, num_subcores=_NS)
  out = jax.ShapeDtypeStruct((batch, _D), jnp.float32)
  out_s = jax.ShapeDtypeStruct((batch, 16), jnp.float32)
  buf = pltpu.VMEM((_CHUNK, _D), jnp.float32)
  k = pl.kernel(
      functools.partial(_sc_gather_body, batch=batch),
      out_type=(out, out, out_s),
      mesh=mesh,
      scratch_types=[
          pltpu.VMEM((16,), jnp.int32),
          pltpu.VMEM((rows_per_w,), jnp.int32),
          pltpu.VMEM((rows_per_w,), jnp.int32),
          pltpu.VMEM((1, _D + 32), jnp.float32),
          pltpu.VMEM((rows_per_w, 16), jnp.float32),
      ] + [buf] * 8 + [pltpu.SemaphoreType.DMA] * 12,
  )
  return k


def _sc_gather(base, uidx, iidx, t_umf, t_imf, t_umlp, t_imlp, Wo, batch):
  return _sc_gather_kernel(batch)(base, uidx, iidx, t_umf, t_imf, t_umlp,
                                  t_imlp, Wo)


_BLK = 1024


def _tc_mlp_body(umlp, imlp, s, w1, b1, w2, b2, w3, b3, wo, bo, out):
  cT = (((1,), (1,)), ((), ()))
  cN = (((1,), (0,)), ((), ()))
  h = lax.dot_general(w1[:, :_D], umlp[...], cT,
                      preferred_element_type=jnp.float32)
  h += lax.dot_general(w1[:, _D:], imlp[...], cT,
                       preferred_element_type=jnp.float32)
  h = jnp.maximum(h + b1[...][:, None], 0.0)
  h = jnp.maximum(
      lax.dot_general(w2[...], h, cN, preferred_element_type=jnp.float32)
      + b2[...][:, None], 0.0)
  h = jnp.maximum(
      lax.dot_general(w3[...], h, cN, preferred_element_type=jnp.float32)
      + b3[...][:, None], 0.0)
  logit = lax.dot_general(wo[:, _D:], h, cN,
                          preferred_element_type=jnp.float32)
  ones16 = jnp.ones((1, 16), dtype=jnp.float32)
  logit += lax.dot_general(ones16, s[...], cT,
                           preferred_element_type=jnp.float32)
  logit += bo[0]
  out[...] = (1.0 / (1.0 + jnp.exp(-logit)))[0, :]


def _tc_mlp(umlp, imlp, s, W1, b1, W2, b2, W3, b3, Wo, bo, batch):
  n_blk = batch // _BLK
  batch_spec = pl.BlockSpec((_BLK, _D), lambda i: (i, 0))
  s_spec = pl.BlockSpec((_BLK, 16), lambda i: (i, 0))
  full = lambda shape: pl.BlockSpec(shape, lambda i: tuple(0 for _ in shape))
  return pl.pallas_call(
      _tc_mlp_body,
      grid=(n_blk,),
      in_specs=[
          batch_spec, batch_spec, s_spec,
          full((_D, 2 * _D)), full((_D,)),
          full((64, _D)), full((64,)),
          full((32, 64)), full((32,)),
          full((1, _D + 32)), full((1,)),
      ],
      out_specs=pl.BlockSpec((_BLK,), lambda i: (i,)),
      out_shape=jax.ShapeDtypeStruct((batch,), jnp.float32),
  )(umlp, imlp, s, W1, b1, W2, b2, W3, b3, Wo, bo)


_N_SPLIT = 1


@jax.jit
def kernel(user_indices, item_indices, user_emb_mf, item_emb_mf,
           user_emb_mlp, item_emb_mlp, W1, b1, W2, b2, W3, b3, Wo, bo):
  half = _BATCH // _N_SPLIT
  outs = []
  for h in range(_N_SPLIT):
    base = jnp.full((16,), h * half, dtype=jnp.int32)
    umlp, imlp, s = _sc_gather(base, user_indices, item_indices, user_emb_mf,
                               item_emb_mf, user_emb_mlp, item_emb_mlp, Wo,
                               half)
    outs.append(_tc_mlp(umlp, imlp, s, W1, b1, W2, b2, W3, b3, Wo, bo, half))
  return jnp.concatenate(outs)


# speedup vs baseline: 1.0347x; 1.0210x over previous
import functools

import jax
import jax.numpy as jnp
from jax import lax
from jax.experimental import pallas as pl
from jax.experimental.pallas import tpu as pltpu
from jax.experimental.pallas import tpu_sc as plsc

_NC = 2
_NS = 16
_NW = _NC * _NS

_BATCH = 16384
_D = 128
_CHUNK = 32


def _sc_gather_body(base_hbm, uidx_hbm, iidx_hbm, t_umf, t_imf, t_umlp,
                    t_imlp, wo_hbm,
                    o_umlp, o_imlp, o_s,
                    base_v, uidx_v, iidx_v, wo_v, sdot_v,
                    mf_u0, mf_u1, mf_i0, mf_i1,
                    ml_u0, ml_u1, ml_i0, ml_i1,
                    s_mfu0, s_mfu1, s_mfi0, s_mfi1,
                    s_mlu0, s_mlu1, s_mli0, s_mli1,
                    s_wbu0, s_wbu1, s_wbi0, s_wbi1,
                    batch=None):
  rows_per_w = batch // _NW
  n_chunks = rows_per_w // _CHUNK
  wid = lax.axis_index("s") * _NC + lax.axis_index("c")
  out_base = wid * rows_per_w

  pltpu.sync_copy(base_hbm, base_v)
  base = pl.multiple_of(base_v[...][0], 256)
  pltpu.sync_copy(uidx_hbm.at[pl.ds(base + out_base, rows_per_w)], uidx_v)
  pltpu.sync_copy(iidx_hbm.at[pl.ds(base + out_base, rows_per_w)], iidx_v)
  pltpu.sync_copy(wo_hbm, wo_v)
  w_regs = [wo_v[0, pl.ds(16 * j, 16)] for j in range(_D // 16)]

  def uidx(c):
    return uidx_v.at[pl.ds(c * _CHUNK, _CHUNK)]

  def iidx(c):
    return iidx_v.at[pl.ds(c * _CHUNK, _CHUNK)]

  mf_u, mf_i = (mf_u0, mf_u1), (mf_i0, mf_i1)
  ml_u, ml_i = (ml_u0, ml_u1), (ml_i0, ml_i1)
  smf_u, smf_i = (s_mfu0, s_mfu1), (s_mfi0, s_mfi1)
  sml_u, sml_i = (s_mlu0, s_mlu1), (s_mli0, s_mli1)
  swb_u, swb_i = (s_wbu0, s_wbu1), (s_wbi0, s_wbi1)

  def off(c):
    return pl.multiple_of(c * _CHUNK, _CHUNK)

  def uidx_d(c):
    return uidx_v.at[pl.ds(off(c), _CHUNK)]

  def iidx_d(c):
    return iidx_v.at[pl.ds(off(c), _CHUNK)]

  def start(c, p):
    pltpu.async_copy(t_umf.at[uidx_d(c)], mf_u[p], smf_u[p])
    pltpu.async_copy(t_imf.at[iidx_d(c)], mf_i[p], smf_i[p])
    pltpu.async_copy(t_umlp.at[uidx_d(c)], ml_u[p], sml_u[p])
    pltpu.async_copy(t_imlp.at[iidx_d(c)], ml_i[p], sml_i[p])

  def wait_wb(c, p):
    pltpu.make_async_copy(
        ml_u[p], o_umlp.at[pl.ds(out_base + off(c), _CHUNK)], swb_u[p]).wait()
    pltpu.make_async_copy(
        ml_i[p], o_imlp.at[pl.ds(out_base + off(c), _CHUNK)], swb_i[p]).wait()

  def step(c, p, first, last):
    pltpu.make_async_copy(t_umlp.at[uidx_d(c)], ml_u[p], sml_u[p]).wait()
    pltpu.make_async_copy(t_imlp.at[iidx_d(c)], ml_i[p], sml_i[p]).wait()
    pltpu.async_copy(
        ml_u[p], o_umlp.at[pl.ds(out_base + off(c), _CHUNK)], swb_u[p])
    pltpu.async_copy(
        ml_i[p], o_imlp.at[pl.ds(out_base + off(c), _CHUNK)], swb_i[p])

    pltpu.make_async_copy(t_umf.at[uidx_d(c)], mf_u[p], smf_u[p]).wait()
    pltpu.make_async_copy(t_imf.at[iidx_d(c)], mf_i[p], smf_i[p]).wait()
    ub, ib = mf_u[p], mf_i[p]

    def row_body(r, _):
      acc = ub[r, pl.ds(0, 16)] * ib[r, pl.ds(0, 16)] * w_regs[0]
      for j in range(1, _D // 16):
        acc += ub[r, pl.ds(16 * j, 16)] * ib[r, pl.ds(16 * j, 16)] * w_regs[j]
      sdot_v[off(c) + r, :] = acc
      return 0

    lax.fori_loop(0, _CHUNK, row_body, 0)

    if not last:
      wait_wb(c, p)
      start(c + 2, p)

  start(0, 0)
  start(1, 1)
  n_pairs = n_chunks // 2

  def pair_body(t, _):
    c0 = t * 2

    @pl.when(t < n_pairs - 1)
    def _():
      step(c0, 0, t == 0, False)
      step(c0 + 1, 1, False, False)

    @pl.when(t == n_pairs - 1)
    def _():
      step(c0, 0, False, True)
      step(c0 + 1, 1, False, True)

    return 0

  lax.fori_loop(0, n_pairs, pair_body, 0)

  wait_wb(n_chunks - 2, 0)
  wait_wb(n_chunks - 1, 1)

  pltpu.sync_copy(sdot_v, o_s.at[pl.ds(out_base, rows_per_w)])


@functools.lru_cache(maxsize=None)
def _sc_gather_kernel(batch):
  rows_per_w = batch // _NW
  mesh = plsc.VectorSubcoreMesh(core_axis_name="c", subcore_axis_name="s",
                                num_cores=_NC, num_subcores=_NS)
  out = jax.ShapeDtypeStruct((batch, _D), jnp.float32)
  out_s = jax.ShapeDtypeStruct((batch, 16), jnp.float32)
  buf = pltpu.VMEM((_CHUNK, _D), jnp.float32)
  k = pl.kernel(
      functools.partial(_sc_gather_body, batch=batch),
      out_type=(out, out, out_s),
      mesh=mesh,
      scratch_types=[
          pltpu.VMEM((16,), jnp.int32),
          pltpu.VMEM((rows_per_w,), jnp.int32),
          pltpu.VMEM((rows_per_w,), jnp.int32),
          pltpu.VMEM((1, _D + 32), jnp.float32),
          pltpu.VMEM((rows_per_w, 16), jnp.float32),
      ] + [buf] * 8 + [pltpu.SemaphoreType.DMA] * 12,
  )
  return k


def _sc_gather(base, uidx, iidx, t_umf, t_imf, t_umlp, t_imlp, Wo, batch):
  return _sc_gather_kernel(batch)(base, uidx, iidx, t_umf, t_imf, t_umlp,
                                  t_imlp, Wo)


_BLK = 1024


def _tc_mlp_body(umlp, imlp, s, w1, b1, w2, b2, w3, b3, wo, bo, out):
  cT = (((1,), (1,)), ((), ()))
  cN = (((1,), (0,)), ((), ()))
  h = lax.dot_general(w1[:, :_D], umlp[...], cT,
                      preferred_element_type=jnp.float32)
  h += lax.dot_general(w1[:, _D:], imlp[...], cT,
                       preferred_element_type=jnp.float32)
  h = jnp.maximum(h + b1[...][:, None], 0.0)
  h = jnp.maximum(
      lax.dot_general(w2[...], h, cN, preferred_element_type=jnp.float32)
      + b2[...][:, None], 0.0)
  h = jnp.maximum(
      lax.dot_general(w3[...], h, cN, preferred_element_type=jnp.float32)
      + b3[...][:, None], 0.0)
  logit = lax.dot_general(wo[:, _D:], h, cN,
                          preferred_element_type=jnp.float32)
  ones16 = jnp.ones((1, 16), dtype=jnp.float32)
  logit += lax.dot_general(ones16, s[...], cT,
                           preferred_element_type=jnp.float32)
  logit += bo[0]
  out[...] = (1.0 / (1.0 + jnp.exp(-logit)))[0, :]


def _tc_mlp(umlp, imlp, s, W1, b1, W2, b2, W3, b3, Wo, bo, batch):
  n_blk = batch // _BLK
  batch_spec = pl.BlockSpec((_BLK, _D), lambda i: (i, 0))
  s_spec = pl.BlockSpec((_BLK, 16), lambda i: (i, 0))
  full = lambda shape: pl.BlockSpec(shape, lambda i: tuple(0 for _ in shape))
  return pl.pallas_call(
      _tc_mlp_body,
      grid=(n_blk,),
      in_specs=[
          batch_spec, batch_spec, s_spec,
          full((_D, 2 * _D)), full((_D,)),
          full((64, _D)), full((64,)),
          full((32, 64)), full((32,)),
          full((1, _D + 32)), full((1,)),
      ],
      out_specs=pl.BlockSpec((_BLK,), lambda i: (i,)),
      out_shape=jax.ShapeDtypeStruct((batch,), jnp.float32),
  )(umlp, imlp, s, W1, b1, W2, b2, W3, b3, Wo, bo)


_N_SPLIT = 1


@jax.jit
def kernel(user_indices, item_indices, user_emb_mf, item_emb_mf,
           user_emb_mlp, item_emb_mlp, W1, b1, W2, b2, W3, b3, Wo, bo):
  half = _BATCH // _N_SPLIT
  outs = []
  for h in range(_N_SPLIT):
    base = jnp.full((16,), h * half, dtype=jnp.int32)
    umlp, imlp, s = _sc_gather(base, user_indices, item_indices, user_emb_mf,
                               item_emb_mf, user_emb_mlp, item_emb_mlp, Wo,
                               half)
    outs.append(_tc_mlp(umlp, imlp, s, W1, b1, W2, b2, W3, b3, Wo, bo, half))
  return jnp.concatenate(outs)


# speedup vs baseline: 1.1238x; 1.0862x over previous
import functools

import jax
import jax.numpy as jnp
from jax import lax
from jax.experimental import pallas as pl
from jax.experimental.pallas import tpu as pltpu
from jax.experimental.pallas import tpu_sc as plsc

_NC = 2
_NS = 16
_NW = _NC * _NS

_BATCH = 16384
_D = 128
_CHUNK = 32


def _sc_gather_body(base_hbm, uidx_hbm, iidx_hbm, t_umf, t_imf, t_umlp,
                    t_imlp, wo_hbm,
                    o_umlp, o_imlp, o_s,
                    base_v, uidx_v, iidx_v, wo_v, sdot_v,
                    mf_u0, mf_u1, mf_i0, mf_i1,
                    ml_u0, ml_u1, ml_i0, ml_i1,
                    s_mfu0, s_mfu1, s_mfi0, s_mfi1,
                    s_mlu0, s_mlu1, s_mli0, s_mli1,
                    s_wbu0, s_wbu1, s_wbi0, s_wbi1,
                    batch=None):
  rows_per_w = batch // _NW
  n_chunks = rows_per_w // _CHUNK
  wid = lax.axis_index("s") * _NC + lax.axis_index("c")
  out_base = wid * rows_per_w

  pltpu.sync_copy(base_hbm, base_v)
  base = pl.multiple_of(base_v[...][0], 256)
  pltpu.sync_copy(uidx_hbm.at[pl.ds(base + out_base, rows_per_w)], uidx_v)
  pltpu.sync_copy(iidx_hbm.at[pl.ds(base + out_base, rows_per_w)], iidx_v)
  pltpu.sync_copy(wo_hbm, wo_v)
  w_regs = [wo_v[0, pl.ds(16 * j, 16)] for j in range(_D // 16)]

  def uidx(c):
    return uidx_v.at[pl.ds(c * _CHUNK, _CHUNK)]

  def iidx(c):
    return iidx_v.at[pl.ds(c * _CHUNK, _CHUNK)]

  mf_u, mf_i = (mf_u0, mf_u1), (mf_i0, mf_i1)
  ml_u, ml_i = (ml_u0, ml_u1), (ml_i0, ml_i1)
  smf_u, smf_i = (s_mfu0, s_mfu1), (s_mfi0, s_mfi1)
  sml_u, sml_i = (s_mlu0, s_mlu1), (s_mli0, s_mli1)
  swb_u, swb_i = (s_wbu0, s_wbu1), (s_wbi0, s_wbi1)

  def off(c):
    return pl.multiple_of(c * _CHUNK, _CHUNK)

  def uidx_d(c):
    return uidx_v.at[pl.ds(off(c), _CHUNK)]

  def iidx_d(c):
    return iidx_v.at[pl.ds(off(c), _CHUNK)]

  def start(c, p):
    pltpu.async_copy(t_umf.at[uidx_d(c)], mf_u[p], smf_u[p])
    pltpu.async_copy(t_imf.at[iidx_d(c)], mf_i[p], smf_i[p])
    pltpu.async_copy(t_umlp.at[uidx_d(c)], ml_u[p], sml_u[p])
    pltpu.async_copy(t_imlp.at[iidx_d(c)], ml_i[p], sml_i[p])

  def wait_wb(c, p):
    pltpu.make_async_copy(
        ml_u[p], o_umlp.at[pl.ds(out_base + off(c), _CHUNK)], swb_u[p]).wait()
    pltpu.make_async_copy(
        ml_i[p], o_imlp.at[pl.ds(out_base + off(c), _CHUNK)], swb_i[p]).wait()

  def step(c, p, first, last):
    pltpu.make_async_copy(t_umlp.at[uidx_d(c)], ml_u[p], sml_u[p]).wait()
    pltpu.make_async_copy(t_imlp.at[iidx_d(c)], ml_i[p], sml_i[p]).wait()
    pltpu.async_copy(
        ml_u[p], o_umlp.at[pl.ds(out_base + off(c), _CHUNK)], swb_u[p])
    pltpu.async_copy(
        ml_i[p], o_imlp.at[pl.ds(out_base + off(c), _CHUNK)], swb_i[p])

    pltpu.make_async_copy(t_umf.at[uidx_d(c)], mf_u[p], smf_u[p]).wait()
    pltpu.make_async_copy(t_imf.at[iidx_d(c)], mf_i[p], smf_i[p]).wait()
    ub, ib = mf_u[p], mf_i[p]

    def row_body(rr, _):
      for u in range(2):
        r = rr * 2 + u
        acc = ub[r, pl.ds(0, 16)] * ib[r, pl.ds(0, 16)] * w_regs[0]
        for j in range(1, _D // 16):
          acc += (ub[r, pl.ds(16 * j, 16)] * ib[r, pl.ds(16 * j, 16)]
                  * w_regs[j])
        sdot_v[off(c) + r, :] = acc
      return 0

    lax.fori_loop(0, _CHUNK // 2, row_body, 0)

    if not last:
      wait_wb(c, p)
      start(c + 2, p)

  start(0, 0)
  start(1, 1)
  n_pairs = n_chunks // 2

  def pair_body(t, _):
    c0 = t * 2

    @pl.when(t < n_pairs - 1)
    def _():
      step(c0, 0, t == 0, False)
      step(c0 + 1, 1, False, False)

    @pl.when(t == n_pairs - 1)
    def _():
      step(c0, 0, False, True)
      step(c0 + 1, 1, False, True)

    return 0

  lax.fori_loop(0, n_pairs, pair_body, 0)

  wait_wb(n_chunks - 2, 0)
  wait_wb(n_chunks - 1, 1)

  pltpu.sync_copy(sdot_v, o_s.at[pl.ds(out_base, rows_per_w)])


@functools.lru_cache(maxsize=None)
def _sc_gather_kernel(batch):
  rows_per_w = batch // _NW
  mesh = plsc.VectorSubcoreMesh(core_axis_name="c", subcore_axis_name="s",
                                num_cores=_NC, num_subcores=_NS)
  out = jax.ShapeDtypeStruct((batch, _D), jnp.float32)
  out_s = jax.ShapeDtypeStruct((batch, 16), jnp.float32)
  buf = pltpu.VMEM((_CHUNK, _D), jnp.float32)
  k = pl.kernel(
      functools.partial(_sc_gather_body, batch=batch),
      out_type=(out, out, out_s),
      mesh=mesh,
      scratch_types=[
          pltpu.VMEM((16,), jnp.int32),
          pltpu.VMEM((rows_per_w,), jnp.int32),
          pltpu.VMEM((rows_per_w,), jnp.int32),
          pltpu.VMEM((1, _D + 32), jnp.float32),
          pltpu.VMEM((rows_per_w, 16), jnp.float32),
      ] + [buf] * 8 + [pltpu.SemaphoreType.DMA] * 12,
  )
  return k


def _sc_gather(base, uidx, iidx, t_umf, t_imf, t_umlp, t_imlp, Wo, batch):
  return _sc_gather_kernel(batch)(base, uidx, iidx, t_umf, t_imf, t_umlp,
                                  t_imlp, Wo)


_BLK = 2048


def _tc_mlp_body(umlp, imlp, s, w1, b1, w2, b2, w3, b3, wo, bo, out):
  cT = (((1,), (1,)), ((), ()))
  cN = (((1,), (0,)), ((), ()))
  h = lax.dot_general(w1[:, :_D], umlp[...], cT,
                      preferred_element_type=jnp.float32)
  h += lax.dot_general(w1[:, _D:], imlp[...], cT,
                       preferred_element_type=jnp.float32)
  h = jnp.maximum(h + b1[...][:, None], 0.0)
  h = jnp.maximum(
      lax.dot_general(w2[...], h, cN, preferred_element_type=jnp.float32)
      + b2[...][:, None], 0.0)
  h = jnp.maximum(
      lax.dot_general(w3[...], h, cN, preferred_element_type=jnp.float32)
      + b3[...][:, None], 0.0)
  logit = lax.dot_general(wo[:, _D:], h, cN,
                          preferred_element_type=jnp.float32)
  ones16 = jnp.ones((1, 16), dtype=jnp.float32)
  logit += lax.dot_general(ones16, s[...], cT,
                           preferred_element_type=jnp.float32)
  logit += bo[0]
  out[...] = (1.0 / (1.0 + jnp.exp(-logit)))[0, :]


def _tc_mlp(umlp, imlp, s, W1, b1, W2, b2, W3, b3, Wo, bo, batch):
  n_blk = batch // _BLK
  batch_spec = pl.BlockSpec((_BLK, _D), lambda i: (i, 0))
  s_spec = pl.BlockSpec((_BLK, 16), lambda i: (i, 0))
  full = lambda shape: pl.BlockSpec(shape, lambda i: tuple(0 for _ in shape))
  return pl.pallas_call(
      _tc_mlp_body,
      grid=(n_blk,),
      in_specs=[
          batch_spec, batch_spec, s_spec,
          full((_D, 2 * _D)), full((_D,)),
          full((64, _D)), full((64,)),
          full((32, 64)), full((32,)),
          full((1, _D + 32)), full((1,)),
      ],
      out_specs=pl.BlockSpec((_BLK,), lambda i: (i,)),
      out_shape=jax.ShapeDtypeStruct((batch,), jnp.float32),
  )(umlp, imlp, s, W1, b1, W2, b2, W3, b3, Wo, bo)


_N_SPLIT = 1


@jax.jit
def kernel(user_indices, item_indices, user_emb_mf, item_emb_mf,
           user_emb_mlp, item_emb_mlp, W1, b1, W2, b2, W3, b3, Wo, bo):
  half = _BATCH // _N_SPLIT
  outs = []
  for h in range(_N_SPLIT):
    base = jnp.full((16,), h * half, dtype=jnp.int32)
    umlp, imlp, s = _sc_gather(base, user_indices, item_indices, user_emb_mf,
                               item_emb_mf, user_emb_mlp, item_emb_mlp, Wo,
                               half)
    outs.append(_tc_mlp(umlp, imlp, s, W1, b1, W2, b2, W3, b3, Wo, bo, half))
  return jnp.concatenate(outs)
